# Initial kernel scaffold; baseline (speedup 1.0000x reference)
import jax
import jax.numpy as jnp
from jax.experimental import pallas as pl


def kernel(x, edge_index, edge_attr, batch, W_vert, W_edge, W_conv, b_conv, W_h1, b_h1, W_h2, b_h2):
    raise NotImplementedError("write your pallas kernel here")


# speedup vs baseline: 3.1423x; 3.1423x over previous
import functools

import jax
import jax.numpy as jnp
from jax import lax
from jax.experimental import pallas as pl
from jax.experimental.pallas import tpu as pltpu
from jax.experimental.pallas import tpu_sc as plsc

N = 50000
E = 800000
D = 64
H = 32
G = 64
L = 9
EPS = 1e-05
NEG = -1e30

IDXW = 128
CHUNK = 256
SUB = CHUNK // IDXW
TILES = 16
E_PAD = 802816
CHUNKS = E_PAD // (TILES * CHUNK)
ROWS_PER_TILE = E_PAD // (TILES * IDXW)
N_PAD = 50176
NPT = N_PAD // TILES
ZROWS = 56

RBLK = 2000
NBLKS = N // RBLK
EBLK = 8192


def _sc_message(h0, h1, ea0, ea1, src2d, dst2d):
    mesh = plsc.VectorSubcoreMesh(core_axis_name="c", subcore_axis_name="s")

    @functools.partial(
        pl.kernel,
        mesh=mesh,
        compiler_params=pltpu.CompilerParams(use_tc_tiling_on_sc=False),
        out_type=[jax.ShapeDtypeStruct((N_PAD, H), jnp.float32),
                  jax.ShapeDtypeStruct((N_PAD, H), jnp.float32)],
        scratch_types=[
            pltpu.VMEM((SUB, IDXW), jnp.int32),
            pltpu.VMEM((SUB, IDXW), jnp.int32),
            pltpu.VMEM((CHUNK, H), jnp.float32),
            pltpu.VMEM((CHUNK, H), jnp.float32),
            pltpu.VMEM((ZROWS, H), jnp.float32),
            pltpu.VMEM_SHARED((N_PAD, H), jnp.float32),
            pltpu.SemaphoreType.DMA,
        ],
    )
    def k(h0_hbm, h1_hbm, ea0_hbm, ea1_hbm, src_hbm, dst_hbm,
          out0, out1, sidx, didx, gbuf, ebuf, zbuf, accum, sem):
        c = lax.axis_index("c")
        s = lax.axis_index("s")

        def zb(r, carry):
            z = jnp.zeros((16,), jnp.float32)
            zbuf[r, pl.ds(0, 16)] = z
            zbuf[r, pl.ds(16, 16)] = z
            return carry
        lax.fori_loop(0, ZROWS, zb, 0)

        def za(q, carry):
            pltpu.sync_copy(zbuf, accum.at[pl.ds(s * NPT + q * ZROWS, ZROWS)])
            return carry
        lax.fori_loop(0, NPT // ZROWS, za, 0)
        plsc.subcore_barrier()

        def chunk_body(j, carry):
            rowbase = s * ROWS_PER_TILE + j * SUB
            pltpu.sync_copy(src_hbm.at[pl.ds(rowbase, SUB)], sidx)
            pltpu.sync_copy(dst_hbm.at[pl.ds(rowbase, SUB)], didx)
            ebase = rowbase * IDXW

            @pl.when(c == 0)
            def _():
                pltpu.sync_copy(ea0_hbm.at[pl.ds(ebase, CHUNK)], ebuf)
                cps = [pltpu.async_copy(h0_hbm.at[sidx.at[jj]],
                                        gbuf.at[pl.ds(jj * IDXW, IDXW)], sem)
                       for jj in range(SUB)]
                for cp in cps:
                    cp.wait()

            @pl.when(c == 1)
            def _():
                pltpu.sync_copy(ea1_hbm.at[pl.ds(ebase, CHUNK)], ebuf)
                cps = [pltpu.async_copy(h1_hbm.at[sidx.at[jj]],
                                        gbuf.at[pl.ds(jj * IDXW, IDXW)], sem)
                       for jj in range(SUB)]
                for cp in cps:
                    cp.wait()

            def rb(it, carry2):
                base = it * 8
                for u in range(8):
                    r = base + u
                    for half in range(2):
                        sl = pl.ds(half * 16, 16)
                        gbuf[r, sl] = jnp.maximum(gbuf[r, sl] + ebuf[r, sl], 0.0)
                return carry2
            lax.fori_loop(0, CHUNK // 8, rb, 0)

            for jj in range(SUB):
                pltpu.sync_copy(gbuf.at[pl.ds(jj * IDXW, IDXW)],
                                accum.at[didx.at[jj]], add=True)
            return carry
        lax.fori_loop(0, CHUNKS, chunk_body, 0)
        plsc.subcore_barrier()

        def rb2(q, carry):
            base = s * NPT + q * ZROWS

            @pl.when(c == 0)
            def _():
                pltpu.sync_copy(accum.at[pl.ds(base, ZROWS)], zbuf)
                pltpu.sync_copy(zbuf, out0.at[pl.ds(base, ZROWS)])

            @pl.when(c == 1)
            def _():
                pltpu.sync_copy(accum.at[pl.ds(base, ZROWS)], zbuf)
                pltpu.sync_copy(zbuf, out1.at[pl.ds(base, ZROWS)])
            return carry
        lax.fori_loop(0, NPT // ZROWS, rb2, 0)

    return k(h0, h1, ea0, ea1, src2d, dst2d)


def _embed_body(x_ref, wv_ref, h0_ref, h1_ref):
    h = jnp.dot(x_ref[...], wv_ref[...], preferred_element_type=jnp.float32)
    h0_ref[...] = h[:, :H]
    h1_ref[...] = h[:, H:]


def _tc_embed(x, W_vert):
    return pl.pallas_call(
        _embed_body,
        grid=(NBLKS,),
        in_specs=[pl.BlockSpec((RBLK, 13), lambda i: (i, 0)),
                  pl.BlockSpec((13, D), lambda i: (0, 0))],
        out_specs=[pl.BlockSpec((RBLK, H), lambda i: (i, 0)),
                   pl.BlockSpec((RBLK, H), lambda i: (i, 0))],
        out_shape=[jax.ShapeDtypeStruct((N, H), jnp.float32),
                   jax.ShapeDtypeStruct((N, H), jnp.float32)],
    )(x, W_vert)


def _ea_body(eattr_ref, we_ref, ea0_ref, ea1_ref):
    i = pl.program_id(0)
    ea = jnp.dot(eattr_ref[...], we_ref[...], preferred_element_type=jnp.float32)
    rows = i * EBLK + lax.broadcasted_iota(jnp.int32, (EBLK, 1), 0)
    ea = jnp.where(rows < E, ea, NEG)
    ea0_ref[...] = ea[:, :H]
    ea1_ref[...] = ea[:, H:]


def _tc_edge_embed(eattr_pad, W_edge):
    return pl.pallas_call(
        _ea_body,
        grid=(E_PAD // EBLK,),
        in_specs=[pl.BlockSpec((EBLK, 4), lambda i: (i, 0)),
                  pl.BlockSpec((4, D), lambda i: (0, 0))],
        out_specs=[pl.BlockSpec((EBLK, H), lambda i: (i, 0)),
                   pl.BlockSpec((EBLK, H), lambda i: (i, 0))],
        out_shape=[jax.ShapeDtypeStruct((E_PAD, H), jnp.float32),
                   jax.ShapeDtypeStruct((E_PAD, H), jnp.float32)],
    )(eattr_pad, W_edge)


def _layer_body(h0_ref, h1_ref, a0_ref, a1_ref, w_ref, b_ref, o0_ref, o1_ref):
    hb = jnp.concatenate([h0_ref[...], h1_ref[...]], axis=1)
    a = jnp.concatenate([a0_ref[...], a1_ref[...]], axis=1)
    y = jnp.dot((1.0 + EPS) * hb + a, w_ref[...],
                preferred_element_type=jnp.float32) + b_ref[...]
    y = jnp.maximum(y, 0.0) + hb
    o0_ref[...] = y[:, :H]
    o1_ref[...] = y[:, H:]


def _tc_layer(h0, h1, a0, a1, w, b):
    return pl.pallas_call(
        _layer_body,
        grid=(NBLKS,),
        in_specs=[pl.BlockSpec((RBLK, H), lambda i: (i, 0)),
                  pl.BlockSpec((RBLK, H), lambda i: (i, 0)),
                  pl.BlockSpec((RBLK, H), lambda i: (i, 0)),
                  pl.BlockSpec((RBLK, H), lambda i: (i, 0)),
                  pl.BlockSpec((D, D), lambda i: (0, 0)),
                  pl.BlockSpec((1, D), lambda i: (0, 0))],
        out_specs=[pl.BlockSpec((RBLK, H), lambda i: (i, 0)),
                   pl.BlockSpec((RBLK, H), lambda i: (i, 0))],
        out_shape=[jax.ShapeDtypeStruct((N, H), jnp.float32),
                   jax.ShapeDtypeStruct((N, H), jnp.float32)],
    )(h0, h1, a0, a1, w, b)


def _head_body(b3_ref, h0_ref, h1_ref, wh1_ref, bh1_ref, wh2_ref, bh2_ref,
               out_ref, sums, cnt):
    i = pl.program_id(0)

    @pl.when(i == 0)
    def _():
        sums[...] = jnp.zeros_like(sums)
        cnt[...] = jnp.zeros_like(cnt)

    hb = jnp.concatenate([h0_ref[...], h1_ref[...]], axis=1)
    bvec = b3_ref[...].reshape(1, RBLK)
    onehot = (lax.broadcasted_iota(jnp.int32, (G, RBLK), 0) == bvec
              ).astype(jnp.float32)
    sums[...] += jnp.dot(onehot, hb, preferred_element_type=jnp.float32)
    cnt[...] += jnp.sum(onehot, axis=1, keepdims=True)

    @pl.when(i == NBLKS - 1)
    def _():
        pooled = sums[...] / jnp.maximum(cnt[...], 1.0)
        z1 = jnp.dot(pooled, wh1_ref[...],
                     preferred_element_type=jnp.float32) + bh1_ref[...]
        z1 = 0.5 * z1 * (1.0 + lax.erf(z1 * (2.0 ** -0.5)))
        out_ref[...] = jnp.dot(z1, wh2_ref[...],
                               preferred_element_type=jnp.float32) + bh2_ref[...]


def _tc_head(batch3, h0, h1, W_h1, b_h1, W_h2, b_h2):
    return pl.pallas_call(
        _head_body,
        grid=(NBLKS,),
        in_specs=[pl.BlockSpec((1, 1, RBLK), lambda i: (i, 0, 0)),
                  pl.BlockSpec((RBLK, H), lambda i: (i, 0)),
                  pl.BlockSpec((RBLK, H), lambda i: (i, 0)),
                  pl.BlockSpec((D, 512), lambda i: (0, 0)),
                  pl.BlockSpec((1, 512), lambda i: (0, 0)),
                  pl.BlockSpec((512, 1), lambda i: (0, 0)),
                  pl.BlockSpec((1, 1), lambda i: (0, 0))],
        out_specs=pl.BlockSpec((G, 1), lambda i: (0, 0)),
        out_shape=jax.ShapeDtypeStruct((G, 1), jnp.float32),
        scratch_shapes=[pltpu.VMEM((G, D), jnp.float32),
                        pltpu.VMEM((G, 1), jnp.float32)],
    )(batch3, h0, h1, W_h1, b_h1, W_h2, b_h2)


def kernel(x, edge_index, edge_attr, batch, W_vert, W_edge, W_conv, b_conv,
           W_h1, b_h1, W_h2, b_h2):
    src = edge_index[0]
    dst = edge_index[1]
    pad = E_PAD - E
    src2d = jnp.pad(src, (0, pad)).reshape(E_PAD // IDXW, IDXW)
    dst2d = jnp.pad(dst, (0, pad)).reshape(E_PAD // IDXW, IDXW)
    eattr_pad = jnp.pad(edge_attr, ((0, pad), (0, 0)))

    ea0, ea1 = _tc_edge_embed(eattr_pad, W_edge)
    h0, h1 = _tc_embed(x, W_vert)

    for i in range(L):
        a0, a1 = _sc_message(h0, h1, ea0, ea1, src2d, dst2d)
        h0, h1 = _tc_layer(h0, h1, a0, a1, W_conv[i],
                           b_conv[i].reshape(1, D))

    batch3 = batch.reshape(NBLKS, 1, RBLK)
    return _tc_head(batch3, h0, h1, W_h1, b_h1.reshape(1, 512),
                    W_h2, b_h2.reshape(1, 1))


# speedup vs baseline: 4.8445x; 1.5417x over previous
import functools

import jax
import jax.numpy as jnp
from jax import lax
from jax.experimental import pallas as pl
from jax.experimental.pallas import tpu as pltpu
from jax.experimental.pallas import tpu_sc as plsc

N = 50000
E = 800000
D = 64
H = 32
G = 64
L = 9
EPS = 1e-05
NEG = -1e30

IDXW = 128
CHUNK = 128
TILES = 16
E_PAD = 802816
SUPER = 14
CHUNKS = E_PAD // (TILES * CHUNK)
SUPERS = CHUNKS // SUPER
ROWS_PER_TILE = E_PAD // (TILES * IDXW)
N_PAD = 50176
DUMP = N
NPT = N_PAD // TILES
ZROWS = 56

RBLK = 2000
NBLKS = N // RBLK
EBLK = 8192


def _sc_message(h0, h1, ea0, ea1, src2d, dst2d):
    mesh = plsc.VectorSubcoreMesh(core_axis_name="c", subcore_axis_name="s")

    @functools.partial(
        pl.kernel,
        mesh=mesh,
        compiler_params=pltpu.CompilerParams(use_tc_tiling_on_sc=False),
        out_type=[jax.ShapeDtypeStruct((N_PAD, H), jnp.float32),
                  jax.ShapeDtypeStruct((N_PAD, H), jnp.float32)],
        scratch_types=[
            pltpu.VMEM((SUPER, IDXW), jnp.int32),
            pltpu.VMEM((SUPER, IDXW), jnp.int32),
            pltpu.VMEM((CHUNK, H), jnp.float32),
            pltpu.VMEM((CHUNK, H), jnp.float32),
            pltpu.VMEM((CHUNK, H), jnp.float32),
            pltpu.VMEM((CHUNK, H), jnp.float32),
            pltpu.VMEM((ZROWS, H), jnp.float32),
            pltpu.VMEM_SHARED((N_PAD, H), jnp.float32),
            pltpu.SemaphoreType.DMA,
            pltpu.SemaphoreType.DMA,
        ],
    )
    def k(h0_hbm, h1_hbm, ea0_hbm, ea1_hbm, src_hbm, dst_hbm,
          out0, out1, sidx, didx, g0b, g1b, e0b, e1b, zbuf, accum, sem0, sem1):
        c = lax.axis_index("c")
        s = lax.axis_index("s")
        gb = (g0b, g1b)
        eb = (e0b, e1b)
        sems = (sem0, sem1)

        def zb(r, carry):
            z = jnp.zeros((16,), jnp.float32)
            zbuf[r, pl.ds(0, 16)] = z
            zbuf[r, pl.ds(16, 16)] = z
            return carry
        lax.fori_loop(0, ZROWS, zb, 0)

        def za(q, carry):
            pltpu.sync_copy(zbuf, accum.at[pl.ds(s * NPT + q * ZROWS, ZROWS)])
            return carry
        lax.fori_loop(0, NPT // ZROWS, za, 0)
        plsc.subcore_barrier()

        def run(h_hbm, ea_hbm, out_hbm):
            def super_body(sj, carry):
                base = s * ROWS_PER_TILE + sj * SUPER
                pltpu.sync_copy(src_hbm.at[pl.ds(base, SUPER)], sidx)
                pltpu.sync_copy(dst_hbm.at[pl.ds(base, SUPER)], didx)

                def issue(cc, b):
                    erow = (base + cc) * IDXW
                    hg = pltpu.async_copy(h_hbm.at[sidx.at[cc]], gb[b], sems[b])
                    he = pltpu.async_copy(ea_hbm.at[pl.ds(erow, CHUNK)],
                                          eb[b], sems[b])
                    return (hg, he)

                hnd = {0: issue(0, 0)}
                for cc in range(SUPER):
                    b = cc & 1
                    if cc + 1 < SUPER:
                        hnd[cc + 1] = issue(cc + 1, 1 - b)
                    for hh in hnd.pop(cc):
                        hh.wait()
                    gbuf, ebuf = gb[b], eb[b]

                    def rb(it, carry2, gbuf=gbuf, ebuf=ebuf):
                        rbase = it * 8
                        for u in range(8):
                            r = rbase + u
                            for half in range(2):
                                sl = pl.ds(half * 16, 16)
                                gbuf[r, sl] = jnp.maximum(
                                    gbuf[r, sl] + ebuf[r, sl], 0.0)
                        return carry2
                    lax.fori_loop(0, CHUNK // 8, rb, 0)
                    pltpu.sync_copy(gbuf, accum.at[didx.at[cc]], add=True)
                return carry
            lax.fori_loop(0, SUPERS, super_body, 0)
            plsc.subcore_barrier()

            def rb2(q, carry):
                rbase = s * NPT + q * ZROWS
                pltpu.sync_copy(accum.at[pl.ds(rbase, ZROWS)], zbuf)
                pltpu.sync_copy(zbuf, out_hbm.at[pl.ds(rbase, ZROWS)])
                return carry
            lax.fori_loop(0, NPT // ZROWS, rb2, 0)

        @pl.when(c == 0)
        def _():
            run(h0_hbm, ea0_hbm, out0)

        @pl.when(c == 1)
        def _():
            run(h1_hbm, ea1_hbm, out1)

    return k(h0, h1, ea0, ea1, src2d, dst2d)


def _embed_body(x_ref, wv_ref, h0_ref, h1_ref):
    h = jnp.dot(x_ref[...], wv_ref[...], preferred_element_type=jnp.float32)
    h0_ref[...] = h[:, :H]
    h1_ref[...] = h[:, H:]


def _tc_embed(x, W_vert):
    return pl.pallas_call(
        _embed_body,
        grid=(NBLKS,),
        in_specs=[pl.BlockSpec((RBLK, 13), lambda i: (i, 0)),
                  pl.BlockSpec((13, D), lambda i: (0, 0))],
        out_specs=[pl.BlockSpec((RBLK, H), lambda i: (i, 0)),
                   pl.BlockSpec((RBLK, H), lambda i: (i, 0))],
        out_shape=[jax.ShapeDtypeStruct((N, H), jnp.float32),
                   jax.ShapeDtypeStruct((N, H), jnp.float32)],
    )(x, W_vert)


def _ea_body(eattr_ref, we_ref, ea0_ref, ea1_ref):
    ea = jnp.dot(eattr_ref[...], we_ref[...], preferred_element_type=jnp.float32)
    ea0_ref[...] = ea[:, :H]
    ea1_ref[...] = ea[:, H:]


def _tc_edge_embed(eattr_pad, W_edge):
    return pl.pallas_call(
        _ea_body,
        grid=(E_PAD // EBLK,),
        in_specs=[pl.BlockSpec((EBLK, 4), lambda i: (i, 0)),
                  pl.BlockSpec((4, D), lambda i: (0, 0))],
        out_specs=[pl.BlockSpec((EBLK, H), lambda i: (i, 0)),
                   pl.BlockSpec((EBLK, H), lambda i: (i, 0))],
        out_shape=[jax.ShapeDtypeStruct((E_PAD, H), jnp.float32),
                   jax.ShapeDtypeStruct((E_PAD, H), jnp.float32)],
    )(eattr_pad, W_edge)


def _layer_body(h0_ref, h1_ref, a0_ref, a1_ref, w_ref, b_ref, o0_ref, o1_ref):
    hb = jnp.concatenate([h0_ref[...], h1_ref[...]], axis=1)
    a = jnp.concatenate([a0_ref[...], a1_ref[...]], axis=1)
    y = jnp.dot((1.0 + EPS) * hb + a, w_ref[...],
                preferred_element_type=jnp.float32) + b_ref[...]
    y = jnp.maximum(y, 0.0) + hb
    o0_ref[...] = y[:, :H]
    o1_ref[...] = y[:, H:]


def _tc_layer(h0, h1, a0, a1, w, b):
    return pl.pallas_call(
        _layer_body,
        grid=(NBLKS,),
        in_specs=[pl.BlockSpec((RBLK, H), lambda i: (i, 0)),
                  pl.BlockSpec((RBLK, H), lambda i: (i, 0)),
                  pl.BlockSpec((RBLK, H), lambda i: (i, 0)),
                  pl.BlockSpec((RBLK, H), lambda i: (i, 0)),
                  pl.BlockSpec((D, D), lambda i: (0, 0)),
                  pl.BlockSpec((1, D), lambda i: (0, 0))],
        out_specs=[pl.BlockSpec((RBLK, H), lambda i: (i, 0)),
                   pl.BlockSpec((RBLK, H), lambda i: (i, 0))],
        out_shape=[jax.ShapeDtypeStruct((N, H), jnp.float32),
                   jax.ShapeDtypeStruct((N, H), jnp.float32)],
    )(h0, h1, a0, a1, w, b)


def _head_body(b3_ref, h0_ref, h1_ref, wh1_ref, bh1_ref, wh2_ref, bh2_ref,
               out_ref, sums, cnt):
    i = pl.program_id(0)

    @pl.when(i == 0)
    def _():
        sums[...] = jnp.zeros_like(sums)
        cnt[...] = jnp.zeros_like(cnt)

    hb = jnp.concatenate([h0_ref[...], h1_ref[...]], axis=1)
    bvec = b3_ref[...].reshape(1, RBLK)
    onehot = (lax.broadcasted_iota(jnp.int32, (G, RBLK), 0) == bvec
              ).astype(jnp.float32)
    sums[...] += jnp.dot(onehot, hb, preferred_element_type=jnp.float32)
    cnt[...] += jnp.sum(onehot, axis=1, keepdims=True)

    @pl.when(i == NBLKS - 1)
    def _():
        pooled = sums[...] / jnp.maximum(cnt[...], 1.0)
        z1 = jnp.dot(pooled, wh1_ref[...],
                     preferred_element_type=jnp.float32) + bh1_ref[...]
        z1 = 0.5 * z1 * (1.0 + lax.erf(z1 * (2.0 ** -0.5)))
        out_ref[...] = jnp.dot(z1, wh2_ref[...],
                               preferred_element_type=jnp.float32) + bh2_ref[...]


def _tc_head(batch3, h0, h1, W_h1, b_h1, W_h2, b_h2):
    return pl.pallas_call(
        _head_body,
        grid=(NBLKS,),
        in_specs=[pl.BlockSpec((1, 1, RBLK), lambda i: (i, 0, 0)),
                  pl.BlockSpec((RBLK, H), lambda i: (i, 0)),
                  pl.BlockSpec((RBLK, H), lambda i: (i, 0)),
                  pl.BlockSpec((D, 512), lambda i: (0, 0)),
                  pl.BlockSpec((1, 512), lambda i: (0, 0)),
                  pl.BlockSpec((512, 1), lambda i: (0, 0)),
                  pl.BlockSpec((1, 1), lambda i: (0, 0))],
        out_specs=pl.BlockSpec((G, 1), lambda i: (0, 0)),
        out_shape=jax.ShapeDtypeStruct((G, 1), jnp.float32),
        scratch_shapes=[pltpu.VMEM((G, D), jnp.float32),
                        pltpu.VMEM((G, 1), jnp.float32)],
    )(batch3, h0, h1, W_h1, b_h1, W_h2, b_h2)


def kernel(x, edge_index, edge_attr, batch, W_vert, W_edge, W_conv, b_conv,
           W_h1, b_h1, W_h2, b_h2):
    src = edge_index[0]
    dst = edge_index[1]
    pad = E_PAD - E
    src2d = jnp.pad(src, (0, pad)).reshape(E_PAD // IDXW, IDXW)
    dst2d = jnp.pad(dst, (0, pad), constant_values=DUMP
                    ).reshape(E_PAD // IDXW, IDXW)
    eattr_pad = jnp.pad(edge_attr, ((0, pad), (0, 0)))

    ea0, ea1 = _tc_edge_embed(eattr_pad, W_edge)
    h0, h1 = _tc_embed(x, W_vert)

    for i in range(L):
        a0, a1 = _sc_message(h0, h1, ea0, ea1, src2d, dst2d)
        h0, h1 = _tc_layer(h0, h1, a0, a1, W_conv[i],
                           b_conv[i].reshape(1, D))

    batch3 = batch.reshape(NBLKS, 1, RBLK)
    return _tc_head(batch3, h0, h1, W_h1, b_h1.reshape(1, 512),
                    W_h2, b_h2.reshape(1, 1))


# speedup vs baseline: 5.2593x; 1.0856x over previous
import functools

import jax
import jax.numpy as jnp
from jax import lax
from jax.experimental import pallas as pl
from jax.experimental.pallas import tpu as pltpu
from jax.experimental.pallas import tpu_sc as plsc

N = 50000
E = 800000
D = 64
H = 32
G = 64
L = 9
EPS = 1e-05
NEG = -1e30

IDXW = 128
CHUNK = 128
TILES = 16
E_PAD = 802816
SUPER = 14
CHUNKS = E_PAD // (TILES * CHUNK)
SUPERS = CHUNKS // SUPER
ROWS_PER_TILE = E_PAD // (TILES * IDXW)
N_PAD = 50176
DUMP = N
NPT = N_PAD // TILES
ZROWS = 56

RBLK = 2000
NBLKS = N // RBLK
EBLK = 8192


def _sc_message(h0, h1, ea0, ea1, src2d, dst2d):
    mesh = plsc.VectorSubcoreMesh(core_axis_name="c", subcore_axis_name="s")

    @functools.partial(
        pl.kernel,
        mesh=mesh,
        compiler_params=pltpu.CompilerParams(use_tc_tiling_on_sc=False),
        out_type=[jax.ShapeDtypeStruct((N_PAD, H), jnp.float32),
                  jax.ShapeDtypeStruct((N_PAD, H), jnp.float32)],
        scratch_types=[
            pltpu.VMEM((SUPER, IDXW), jnp.int32),
            pltpu.VMEM((SUPER, IDXW), jnp.int32),
            pltpu.VMEM((CHUNK, H), jnp.float32),
            pltpu.VMEM((CHUNK, H), jnp.float32),
            pltpu.VMEM((CHUNK // 4, 128), jnp.float32),
            pltpu.VMEM((CHUNK // 4, 128), jnp.float32),
            pltpu.VMEM((ZROWS, H), jnp.float32),
            pltpu.VMEM_SHARED((N_PAD, H), jnp.float32),
            pltpu.SemaphoreType.DMA,
            pltpu.SemaphoreType.DMA,
        ],
    )
    def k(h0_hbm, h1_hbm, ea0_hbm, ea1_hbm, src_hbm, dst_hbm,
          out0, out1, sidx, didx, g0b, g1b, e0b, e1b, zbuf, accum, sem0, sem1):
        c = lax.axis_index("c")
        s = lax.axis_index("s")
        gb = (g0b, g1b)
        eb = (e0b, e1b)
        sems = (sem0, sem1)

        def zb(r, carry):
            z = jnp.zeros((16,), jnp.float32)
            zbuf[r, pl.ds(0, 16)] = z
            zbuf[r, pl.ds(16, 16)] = z
            return carry
        lax.fori_loop(0, ZROWS, zb, 0)

        def za(q, carry):
            pltpu.sync_copy(zbuf, accum.at[pl.ds(s * NPT + q * ZROWS, ZROWS)])
            return carry
        lax.fori_loop(0, NPT // ZROWS, za, 0)
        plsc.subcore_barrier()

        def run(h_hbm, ea_hbm, out_hbm):
            def super_body(sj, carry):
                base = s * ROWS_PER_TILE + sj * SUPER
                pltpu.sync_copy(src_hbm.at[pl.ds(base, SUPER)], sidx)
                pltpu.sync_copy(dst_hbm.at[pl.ds(base, SUPER)], didx)

                def issue(cc, b):
                    erow = (base + cc) * (CHUNK // 4)
                    hg = pltpu.async_copy(h_hbm.at[sidx.at[cc]], gb[b], sems[b])
                    he = pltpu.async_copy(ea_hbm.at[pl.ds(erow, CHUNK // 4)],
                                          eb[b], sems[b])
                    return (hg, he)

                hnd = {0: issue(0, 0)}
                for cc in range(SUPER):
                    b = cc & 1
                    if cc + 1 < SUPER:
                        hnd[cc + 1] = issue(cc + 1, 1 - b)
                    for hh in hnd.pop(cc):
                        hh.wait()
                    gbuf, ebuf = gb[b], eb[b]

                    def rb(it, carry2, gbuf=gbuf, ebuf=ebuf):
                        rbase = it * 8
                        erbase = it * 2
                        for u in range(8):
                            r = rbase + u
                            er = erbase + u // 4
                            for half in range(2):
                                sl = pl.ds(half * 16, 16)
                                esl = pl.ds((u % 4) * 32 + half * 16, 16)
                                gbuf[r, sl] = jnp.maximum(
                                    gbuf[r, sl] + ebuf[er, esl], 0.0)
                        return carry2
                    lax.fori_loop(0, CHUNK // 8, rb, 0)
                    pltpu.sync_copy(gbuf, accum.at[didx.at[cc]], add=True)
                return carry
            lax.fori_loop(0, SUPERS, super_body, 0)
            plsc.subcore_barrier()

            def rb2(q, carry):
                rbase = s * NPT + q * ZROWS
                pltpu.sync_copy(accum.at[pl.ds(rbase, ZROWS)], zbuf)
                pltpu.sync_copy(zbuf, out_hbm.at[pl.ds(rbase, ZROWS)])
                return carry
            lax.fori_loop(0, NPT // ZROWS, rb2, 0)

        @pl.when(c == 0)
        def _():
            run(h0_hbm, ea0_hbm, out0)

        @pl.when(c == 1)
        def _():
            run(h1_hbm, ea1_hbm, out1)

    return k(h0, h1, ea0, ea1, src2d, dst2d)


def _embed_body(x_ref, wv_ref, h0_ref, h1_ref):
    h = jnp.dot(x_ref[...], wv_ref[...], preferred_element_type=jnp.float32)
    h0_ref[...] = h[:, :H]
    h1_ref[...] = h[:, H:]


def _tc_embed(x, W_vert):
    return pl.pallas_call(
        _embed_body,
        grid=(NBLKS,),
        in_specs=[pl.BlockSpec((RBLK, 13), lambda i: (i, 0)),
                  pl.BlockSpec((13, D), lambda i: (0, 0))],
        out_specs=[pl.BlockSpec((RBLK, H), lambda i: (i, 0)),
                   pl.BlockSpec((RBLK, H), lambda i: (i, 0))],
        out_shape=[jax.ShapeDtypeStruct((N, H), jnp.float32),
                   jax.ShapeDtypeStruct((N, H), jnp.float32)],
    )(x, W_vert)


def _ea_body(eattr4_ref, we_ref, ea0_ref, ea1_ref):
    att = eattr4_ref[...]
    blk = (lax.broadcasted_iota(jnp.int32, (16, 128), 0) // 4 ==
           lax.broadcasted_iota(jnp.int32, (16, 128), 1) // H)
    for half, o_ref in ((0, ea0_ref), (1, ea1_ref)):
        wh = we_ref[:, half * H:(half + 1) * H]
        wbd = jnp.where(blk, jnp.tile(wh, (4, 4)), 0.0)
        o_ref[...] = jnp.dot(att, wbd, preferred_element_type=jnp.float32)


def _tc_edge_embed(eattr4, W_edge):
    return pl.pallas_call(
        _ea_body,
        grid=(E_PAD // EBLK,),
        in_specs=[pl.BlockSpec((EBLK // 4, 16), lambda i: (i, 0)),
                  pl.BlockSpec((4, D), lambda i: (0, 0))],
        out_specs=[pl.BlockSpec((EBLK // 4, 128), lambda i: (i, 0)),
                   pl.BlockSpec((EBLK // 4, 128), lambda i: (i, 0))],
        out_shape=[jax.ShapeDtypeStruct((E_PAD // 4, 128), jnp.float32),
                   jax.ShapeDtypeStruct((E_PAD // 4, 128), jnp.float32)],
    )(eattr4, W_edge)


def _layer_body(h0_ref, h1_ref, a0_ref, a1_ref, w_ref, b_ref, o0_ref, o1_ref):
    hb = jnp.concatenate([h0_ref[...], h1_ref[...]], axis=1)
    a = jnp.concatenate([a0_ref[...], a1_ref[...]], axis=1)
    y = jnp.dot((1.0 + EPS) * hb + a, w_ref[...],
                preferred_element_type=jnp.float32) + b_ref[...]
    y = jnp.maximum(y, 0.0) + hb
    o0_ref[...] = y[:, :H]
    o1_ref[...] = y[:, H:]


def _tc_layer(h0, h1, a0, a1, w, b):
    return pl.pallas_call(
        _layer_body,
        grid=(NBLKS,),
        in_specs=[pl.BlockSpec((RBLK, H), lambda i: (i, 0)),
                  pl.BlockSpec((RBLK, H), lambda i: (i, 0)),
                  pl.BlockSpec((RBLK, H), lambda i: (i, 0)),
                  pl.BlockSpec((RBLK, H), lambda i: (i, 0)),
                  pl.BlockSpec((D, D), lambda i: (0, 0)),
                  pl.BlockSpec((1, D), lambda i: (0, 0))],
        out_specs=[pl.BlockSpec((RBLK, H), lambda i: (i, 0)),
                   pl.BlockSpec((RBLK, H), lambda i: (i, 0))],
        out_shape=[jax.ShapeDtypeStruct((N, H), jnp.float32),
                   jax.ShapeDtypeStruct((N, H), jnp.float32)],
    )(h0, h1, a0, a1, w, b)


def _head_body(b3_ref, h0_ref, h1_ref, wh1_ref, bh1_ref, wh2_ref, bh2_ref,
               out_ref, sums, cnt):
    i = pl.program_id(0)

    @pl.when(i == 0)
    def _():
        sums[...] = jnp.zeros_like(sums)
        cnt[...] = jnp.zeros_like(cnt)

    hb = jnp.concatenate([h0_ref[...], h1_ref[...]], axis=1)
    bvec = b3_ref[...].reshape(1, RBLK)
    onehot = (lax.broadcasted_iota(jnp.int32, (G, RBLK), 0) == bvec
              ).astype(jnp.float32)
    sums[...] += jnp.dot(onehot, hb, preferred_element_type=jnp.float32)
    cnt[...] += jnp.sum(onehot, axis=1, keepdims=True)

    @pl.when(i == NBLKS - 1)
    def _():
        pooled = sums[...] / jnp.maximum(cnt[...], 1.0)
        z1 = jnp.dot(pooled, wh1_ref[...],
                     preferred_element_type=jnp.float32) + bh1_ref[...]
        z1 = 0.5 * z1 * (1.0 + lax.erf(z1 * (2.0 ** -0.5)))
        out_ref[...] = jnp.dot(z1, wh2_ref[...],
                               preferred_element_type=jnp.float32) + bh2_ref[...]


def _tc_head(batch3, h0, h1, W_h1, b_h1, W_h2, b_h2):
    return pl.pallas_call(
        _head_body,
        grid=(NBLKS,),
        in_specs=[pl.BlockSpec((1, 1, RBLK), lambda i: (i, 0, 0)),
                  pl.BlockSpec((RBLK, H), lambda i: (i, 0)),
                  pl.BlockSpec((RBLK, H), lambda i: (i, 0)),
                  pl.BlockSpec((D, 512), lambda i: (0, 0)),
                  pl.BlockSpec((1, 512), lambda i: (0, 0)),
                  pl.BlockSpec((512, 1), lambda i: (0, 0)),
                  pl.BlockSpec((1, 1), lambda i: (0, 0))],
        out_specs=pl.BlockSpec((G, 1), lambda i: (0, 0)),
        out_shape=jax.ShapeDtypeStruct((G, 1), jnp.float32),
        scratch_shapes=[pltpu.VMEM((G, D), jnp.float32),
                        pltpu.VMEM((G, 1), jnp.float32)],
    )(batch3, h0, h1, W_h1, b_h1, W_h2, b_h2)


def kernel(x, edge_index, edge_attr, batch, W_vert, W_edge, W_conv, b_conv,
           W_h1, b_h1, W_h2, b_h2):
    src = edge_index[0]
    dst = edge_index[1]
    pad = E_PAD - E
    src2d = jnp.pad(src, (0, pad)).reshape(E_PAD // IDXW, IDXW)
    dst2d = jnp.pad(dst, (0, pad), constant_values=DUMP
                    ).reshape(E_PAD // IDXW, IDXW)
    eattr4 = jnp.pad(edge_attr, ((0, pad), (0, 0))).reshape(E_PAD // 4, 16)

    ea0, ea1 = _tc_edge_embed(eattr4, W_edge)
    h0, h1 = _tc_embed(x, W_vert)

    for i in range(L):
        a0, a1 = _sc_message(h0, h1, ea0, ea1, src2d, dst2d)
        h0, h1 = _tc_layer(h0, h1, a0, a1, W_conv[i],
                           b_conv[i].reshape(1, D))

    batch3 = batch.reshape(NBLKS, 1, RBLK)
    return _tc_head(batch3, h0, h1, W_h1, b_h1.reshape(1, 512),
                    W_h2, b_h2.reshape(1, 1))
